# Initial kernel scaffold; baseline (speedup 1.0000x reference)
import jax
import jax.numpy as jnp
from jax.experimental import pallas as pl


def kernel(X, X_merged):
    raise NotImplementedError("write your pallas kernel here")


# speedup vs baseline: 2.9428x; 2.9428x over previous
import functools

import jax
import jax.numpy as jnp
from jax import lax
from jax.experimental import pallas as pl
from jax.experimental.pallas import tpu as pltpu


def _body(x_ref, xm_ref, out_ref, sq_ref, df_ref, *, grid_n, rows, n, k, d, kb):
    i = pl.program_id(0)
    x = x_ref[...]
    sq = jnp.sum(x * x, axis=1)
    r8 = rows // 128
    sq_ref[pl.ds(i * r8, r8), :] = sq.reshape(r8, 128)

    @pl.when(i < kb)
    def _store_diff():
        dxy = x - xm_ref[...]
        df = jnp.sum(dxy * dxy, axis=1)
        df_ref[pl.ds(i * r8, r8), :] = df.reshape(r8, 128)

    @pl.when(i == grid_n - 1)
    def _finalize():
        sq_all = sq_ref[...]
        norms = jnp.sqrt(sq_all)
        total_norm = jnp.sum(norms)
        fro2 = jnp.sum(sq_all)
        bits = lax.bitcast_convert_type(norms, jnp.int32)

        def bisect(_, lohi):
            lo, hi = lohi
            mid = lo + (hi - lo) // 2
            cnt = jnp.sum((bits >= mid).astype(jnp.int32))
            ge = cnt >= k
            return (jnp.where(ge, mid, lo), jnp.where(ge, hi, mid))

        lo, _ = lax.fori_loop(0, 31, bisect, (jnp.int32(0), jnp.int32(0x7F800000)))
        tval = jnp.max(jnp.where(bits <= lo, norms, 0.0))
        gt = bits > lo
        cnt_gt = jnp.sum(gt.astype(jnp.int32))
        sum_gt = jnp.sum(jnp.where(gt, norms, 0.0))
        top_sum = sum_gt + (k - cnt_gt).astype(jnp.float32) * tval

        gamma = top_sum / total_norm
        thr = (1.0 - gamma) * (1.0 - gamma) * fro2
        inv_d = 1.0 / d
        pen1 = jnp.sum(jnp.maximum(df_ref[...] * inv_d - thr, 0.0))
        rows_idx = lax.broadcasted_iota(jnp.int32, sq_all.shape, 0)
        tail = jnp.where(rows_idx >= (kb * rows) // 128,
                         jnp.maximum(sq_all * inv_d - thr, 0.0), 0.0)
        pen2 = jnp.sum(tail)
        out_ref[0, 0] = (pen1 + pen2) / n


def kernel(X, X_merged):
    B, N, D = X.shape
    K = X_merged.shape[1]
    top_k = min(K, N // 2)
    ROWS = 1024
    GRID = N // ROWS
    KB = K // ROWS

    X2 = X.reshape(N, D)
    Xm2 = X_merged.reshape(K, D)

    body = functools.partial(
        _body, grid_n=GRID, rows=ROWS, n=N, k=top_k, d=float(D), kb=KB)
    out = pl.pallas_call(
        body,
        grid=(GRID,),
        in_specs=[
            pl.BlockSpec((ROWS, D), lambda i: (i, 0)),
            pl.BlockSpec((ROWS, D), lambda i: (jnp.minimum(i, KB - 1), 0)),
        ],
        out_specs=pl.BlockSpec(memory_space=pltpu.SMEM),
        out_shape=jax.ShapeDtypeStruct((1, 1), jnp.float32),
        scratch_shapes=[
            pltpu.VMEM((N // 128, 128), jnp.float32),
            pltpu.VMEM((K // 128, 128), jnp.float32),
        ],
    )(X2, Xm2)
    return out.reshape(())
